# baseline (speedup 1.0000x reference)
import functools

import jax
import jax.numpy as jnp
from jax import lax
from jax.experimental import pallas as pl
from jax.experimental.pallas import tpu as pltpu
from jax.experimental.pallas import tpu_sc as plsc

_N = 10000
_E = 640000
_ER = _E // 128
_ER64 = _E // 64


def _node_split(s):
    start = s * 624 + 8 * jnp.minimum(s, 2)
    cnt = jnp.where(s < 2, 632, 624)
    return start, cnt


def _deg_body(dstR, ones128, zeros16, degp_out, idx_v, ones_v, z16_v, hist_sh):
    c = lax.axis_index("c")
    s = lax.axis_index("s")
    w = c * 16 + s
    start = w * 152 + 8 * jnp.minimum(w, 17)
    cnt = jnp.where(w < 17, 160, 152)

    pltpu.sync_copy(dstR.at[pl.ds(start, 152)], idx_v.at[pl.ds(0, 152)])

    @pl.when(w < 17)
    def _():
        pltpu.sync_copy(dstR.at[pl.ds(start + 152, 8)],
                        idx_v.at[pl.ds(152, 8)])

    nstart, ncnt = _node_split(s)
    pltpu.sync_copy(ones128, ones_v)
    pltpu.sync_copy(zeros16, z16_v)
    for z in range(4):
        pltpu.sync_copy(z16_v.at[pl.ds(0, 156)],
                        hist_sh.at[pl.ds(nstart + z * 156, 156)])

    @pl.when(s < 2)
    def _():
        pltpu.sync_copy(z16_v.at[pl.ds(0, 8)],
                        hist_sh.at[pl.ds(nstart + 624, 8)])

    plsc.subcore_barrier()

    def body(r, carry):
        pltpu.sync_copy(ones_v, hist_sh.at[idx_v.at[r]], add=True)
        return carry

    lax.fori_loop(0, cnt, body, 0)
    plsc.subcore_barrier()
    pltpu.sync_copy(hist_sh.at[pl.ds(nstart, 624)],
                    degp_out.at[c, pl.ds(nstart, 624)])

    @pl.when(s < 2)
    def _():
        pltpu.sync_copy(hist_sh.at[pl.ds(nstart + 624, 8)],
                        degp_out.at[c, pl.ds(nstart + 624, 8)])


@functools.cache
def _deg_call():
    return pl.kernel(
        _deg_body,
        out_type=jax.ShapeDtypeStruct((2, _N, 16), jnp.float32),
        mesh=plsc.VectorSubcoreMesh(core_axis_name="c", subcore_axis_name="s"),
        scratch_types=[
            pltpu.VMEM((160, 128), jnp.int32),
            pltpu.VMEM((128, 16), jnp.float32),
            pltpu.VMEM((156, 16), jnp.float32),
            pltpu.VMEM_SHARED((_N, 16), jnp.float32),
        ],
        compiler_params=pltpu.CompilerParams(use_tc_tiling_on_sc=False),
    )


def _zero_out(rows_v, out_sh, zeros128, nstart, s):
    pltpu.sync_copy(zeros128, rows_v)
    for z in range(9):
        pltpu.sync_copy(rows_v.at[pl.ds(0, 64)],
                        out_sh.at[pl.ds(nstart + z * 64, 64)])
    pltpu.sync_copy(rows_v.at[pl.ds(0, 48)],
                    out_sh.at[pl.ds(nstart + 576, 48)])

    @pl.when(s < 2)
    def _():
        pltpu.sync_copy(rows_v.at[pl.ds(0, 8)],
                        out_sh.at[pl.ds(nstart + 624, 8)])


def _edge_chunk(ytab, src_hbm, dst_hbm, src_v, dst_v, rows_v, out_sh,
                base, size):
    pltpu.sync_copy(src_hbm.at[pl.ds(base, size)], src_v.at[pl.ds(0, size)])
    pltpu.sync_copy(dst_hbm.at[pl.ds(base, size)], dst_v.at[pl.ds(0, size)])

    def body(r, carry):
        pltpu.sync_copy(ytab.at[src_v.at[r]], rows_v)
        pltpu.sync_copy(rows_v, out_sh.at[dst_v.at[r]], add=True)
        return carry

    lax.fori_loop(0, size, body, 0)


def _writeout(out_sh, s_out, coff, nstart, s):
    pltpu.sync_copy(out_sh.at[pl.ds(nstart, 624)],
                    s_out.at[pl.ds(coff + nstart, 624)])

    @pl.when(s < 2)
    def _():
        pltpu.sync_copy(out_sh.at[pl.ds(nstart + 624, 8)],
                        s_out.at[pl.ds(coff + nstart + 624, 8)])


def _scatA_body(y1, srcE, dstE, zeros128, s_out,
                src_v, dst_v, rows_v, out_sh):
    c = lax.axis_index("c")
    s = lax.axis_index("s")
    w = c * 16 + s
    start = w * 312 + 8 * jnp.minimum(w, 2)
    nstart, _ = _node_split(s)

    _zero_out(rows_v, out_sh, zeros128, nstart, s)
    plsc.subcore_barrier()

    def full_chunk(k, carry):
        _edge_chunk(y1, srcE, dstE, src_v, dst_v, rows_v, out_sh,
                    start + k * 16, 16)
        return carry

    @pl.when(w < 2)
    def _():
        lax.fori_loop(0, 20, full_chunk, 0)

    @pl.when(w >= 2)
    def _():
        lax.fori_loop(0, 19, full_chunk, 0)
        _edge_chunk(y1, srcE, dstE, src_v, dst_v, rows_v, out_sh,
                    start + 304, 8)

    plsc.subcore_barrier()
    _writeout(out_sh, s_out, c * _N, nstart, s)


@functools.cache
def _make_scatA():
    return pl.kernel(
        _scatA_body,
        out_type=jax.ShapeDtypeStruct((2 * _N, 128), jnp.float32),
        mesh=plsc.VectorSubcoreMesh(core_axis_name="c", subcore_axis_name="s"),
        scratch_types=[
            pltpu.VMEM((16, 64), jnp.int32),
            pltpu.VMEM((16, 64), jnp.int32),
            pltpu.VMEM((64, 128), jnp.float32),
            pltpu.VMEM_SHARED((_N, 128), jnp.float32),
        ],
        compiler_params=pltpu.CompilerParams(use_tc_tiling_on_sc=False),
    )


def _scatB_body(y2f, srcE2, dstE, zeros128, s_out,
                src_v, dst_v, rows_v, out_sh):
    c = lax.axis_index("c")
    s = lax.axis_index("s")
    start, _ = _node_split(s)
    nstart, _ = _node_split(s)

    _zero_out(rows_v, out_sh, zeros128, nstart, s)
    plsc.subcore_barrier()

    def full_chunk(k, carry):
        base = start + k * 16
        pltpu.sync_copy(srcE2.at[c, pl.ds(base, 16)], src_v)
        pltpu.sync_copy(dstE.at[pl.ds(base, 16)], dst_v)

        def body(r, carry2):
            pltpu.sync_copy(y2f.at[src_v.at[r]], rows_v)
            pltpu.sync_copy(rows_v, out_sh.at[dst_v.at[r]], add=True)
            return carry2

        lax.fori_loop(0, 16, body, 0)
        return carry

    lax.fori_loop(0, 39, full_chunk, 0)

    @pl.when(s < 2)
    def _():
        base = start + 624
        pltpu.sync_copy(srcE2.at[c, pl.ds(base, 8)], src_v.at[pl.ds(0, 8)])
        pltpu.sync_copy(dstE.at[pl.ds(base, 8)], dst_v.at[pl.ds(0, 8)])

        def body(r, carry2):
            pltpu.sync_copy(y2f.at[src_v.at[r]], rows_v)
            pltpu.sync_copy(rows_v, out_sh.at[dst_v.at[r]], add=True)
            return carry2

        lax.fori_loop(0, 8, body, 0)

    plsc.subcore_barrier()
    _writeout(out_sh, s_out, c * _N, nstart, s)


@functools.cache
def _make_scatB():
    return pl.kernel(
        _scatB_body,
        out_type=jax.ShapeDtypeStruct((2 * _N, 128), jnp.float32),
        mesh=plsc.VectorSubcoreMesh(core_axis_name="c", subcore_axis_name="s"),
        scratch_types=[
            pltpu.VMEM((16, 64), jnp.int32),
            pltpu.VMEM((16, 64), jnp.int32),
            pltpu.VMEM((64, 128), jnp.float32),
            pltpu.VMEM_SHARED((_N, 128), jnp.float32),
        ],
        compiler_params=pltpu.CompilerParams(use_tc_tiling_on_sc=False),
    )


_R = 1000


def _y1_body(x_ref, w1_ref, degp_ref, y1_ref, dis_ref):
    deg = degp_ref[0][:, 0:1] + degp_ref[1][:, 0:1] + 1.0
    dis = lax.rsqrt(deg)
    xw = jnp.dot(x_ref[...], w1_ref[...], preferred_element_type=jnp.float32)
    y1_ref[...] = xw * dis
    dis_ref[...] = dis


def _y1_call(x, W1, degp):
    return pl.pallas_call(
        _y1_body,
        grid=(_N // _R,),
        in_specs=[
            pl.BlockSpec((_R, 128), lambda i: (i, 0)),
            pl.BlockSpec((128, 128), lambda i: (0, 0)),
            pl.BlockSpec((2, _R, 16), lambda i: (0, i, 0)),
        ],
        out_specs=[
            pl.BlockSpec((_R, 128), lambda i: (i, 0)),
            pl.BlockSpec((_R, 1), lambda i: (i, 0)),
        ],
        out_shape=[
            jax.ShapeDtypeStruct((_N, 128), jnp.float32),
            jax.ShapeDtypeStruct((_N, 1), jnp.float32),
        ],
    )(x, W1, degp)


def _mid_body(s1_ref, y1_ref, dis_ref, b1_ref, w2_ref, y2_ref):
    dis = dis_ref[...]
    pre = s1_ref[0] + s1_ref[1] + y1_ref[...]
    h = jnp.maximum(pre * dis + b1_ref[...], 0.0)
    xw2 = jnp.dot(h, w2_ref[...], preferred_element_type=jnp.float32)
    y2 = xw2 * dis
    y2_ref[0] = y2[:, :128]
    y2_ref[1] = y2[:, 128:]


def _mid_call(s1p, y1, dis, b1r, W2):
    return pl.pallas_call(
        _mid_body,
        grid=(_N // _R,),
        in_specs=[
            pl.BlockSpec((2, _R, 128), lambda i: (0, i, 0)),
            pl.BlockSpec((_R, 128), lambda i: (i, 0)),
            pl.BlockSpec((_R, 1), lambda i: (i, 0)),
            pl.BlockSpec((1, 128), lambda i: (0, 0)),
            pl.BlockSpec((128, 256), lambda i: (0, 0)),
        ],
        out_specs=pl.BlockSpec((2, _R, 128), lambda i: (0, i, 0)),
        out_shape=jax.ShapeDtypeStruct((2, _N, 128), jnp.float32),
    )(s1p, y1, dis, b1r, W2)


def _fin_body(s2_ref, y2_ref, dis_ref, b2_ref, wl_ref, bl_ref, out_ref):
    dis = dis_ref[...]
    pre = jnp.concatenate(
        [s2_ref[0] + y2_ref[0], s2_ref[1] + y2_ref[1]], axis=1)
    h2 = pre * dis + b2_ref[...]
    out_ref[...] = (jnp.dot(h2, wl_ref[...],
                            preferred_element_type=jnp.float32)
                    + bl_ref[...])


def _fin_call(s2f, y2, dis, b2r, WL, bLr):
    return pl.pallas_call(
        _fin_body,
        grid=(_N // _R,),
        in_specs=[
            pl.BlockSpec((2, _R, 128), lambda i: (0, i, 0)),
            pl.BlockSpec((2, _R, 128), lambda i: (0, i, 0)),
            pl.BlockSpec((_R, 1), lambda i: (i, 0)),
            pl.BlockSpec((1, 256), lambda i: (0, 0)),
            pl.BlockSpec((256, 16), lambda i: (0, 0)),
            pl.BlockSpec((1, 16), lambda i: (0, 0)),
        ],
        out_specs=pl.BlockSpec((_R, 16), lambda i: (i, 0)),
        out_shape=jax.ShapeDtypeStruct((_N, 16), jnp.float32),
    )(s2f, y2, dis, b2r, WL, bLr)


def kernel(x, edge_index, W1, b1, W2, b2, WL, bL):
    srcE = edge_index[0].reshape(_ER64, 64)
    dstE = edge_index[1].reshape(_ER64, 64)
    srcE2 = jnp.stack([srcE, srcE + _N])
    dstR = edge_index[1].reshape(_ER, 128)
    ones128 = jnp.ones((128, 16), jnp.float32)
    zeros16 = jnp.zeros((156, 16), jnp.float32)
    zeros128 = jnp.zeros((64, 128), jnp.float32)

    degp = _deg_call()(dstR, ones128, zeros16)
    y1, dis = _y1_call(x, W1, degp)
    s1p = _make_scatA()(y1, srcE, dstE, zeros128).reshape(2, _N, 128)
    y2 = _mid_call(s1p, y1, dis, b1.reshape(1, 128), W2)
    s2f = _make_scatB()(y2.reshape(2 * _N, 128), srcE2, dstE,
                        zeros128).reshape(2, _N, 128)
    return _fin_call(s2f, y2, dis, b2.reshape(1, 256), WL, bL.reshape(1, 16))


# speedup vs baseline: 1.6775x; 1.6775x over previous
import functools

import jax
import jax.numpy as jnp
from jax import lax
from jax.experimental import pallas as pl
from jax.experimental.pallas import tpu as pltpu
from jax.experimental.pallas import tpu_sc as plsc

_N = 10000
_E = 640000
_ER = _E // 128
_ER64 = _E // 64


def _node_split(s):
    start = s * 624 + 8 * jnp.minimum(s, 2)
    cnt = jnp.where(s < 2, 632, 624)
    return start, cnt


def _deg_body(dstR, ones128, zeros16, degp_out, idx_v, ones_v, z16_v, hist_sh):
    c = lax.axis_index("c")
    s = lax.axis_index("s")
    w = c * 16 + s
    start = w * 152 + 8 * jnp.minimum(w, 17)
    cnt = jnp.where(w < 17, 160, 152)

    pltpu.sync_copy(dstR.at[pl.ds(start, 152)], idx_v.at[pl.ds(0, 152)])

    @pl.when(w < 17)
    def _():
        pltpu.sync_copy(dstR.at[pl.ds(start + 152, 8)],
                        idx_v.at[pl.ds(152, 8)])

    nstart, ncnt = _node_split(s)
    pltpu.sync_copy(ones128, ones_v)
    pltpu.sync_copy(zeros16, z16_v)
    for z in range(4):
        pltpu.sync_copy(z16_v.at[pl.ds(0, 156)],
                        hist_sh.at[pl.ds(nstart + z * 156, 156)])

    @pl.when(s < 2)
    def _():
        pltpu.sync_copy(z16_v.at[pl.ds(0, 8)],
                        hist_sh.at[pl.ds(nstart + 624, 8)])

    plsc.subcore_barrier()

    def body(r, carry):
        pltpu.sync_copy(ones_v, hist_sh.at[idx_v.at[r]], add=True)
        return carry

    lax.fori_loop(0, cnt, body, 0)
    plsc.subcore_barrier()
    pltpu.sync_copy(hist_sh.at[pl.ds(nstart, 624)],
                    degp_out.at[c, pl.ds(nstart, 624)])

    @pl.when(s < 2)
    def _():
        pltpu.sync_copy(hist_sh.at[pl.ds(nstart + 624, 8)],
                        degp_out.at[c, pl.ds(nstart + 624, 8)])


@functools.cache
def _deg_call():
    return pl.kernel(
        _deg_body,
        out_type=jax.ShapeDtypeStruct((2, _N, 16), jnp.float32),
        mesh=plsc.VectorSubcoreMesh(core_axis_name="c", subcore_axis_name="s"),
        scratch_types=[
            pltpu.VMEM((160, 128), jnp.int32),
            pltpu.VMEM((128, 16), jnp.float32),
            pltpu.VMEM((156, 16), jnp.float32),
            pltpu.VMEM_SHARED((_N, 16), jnp.float32),
        ],
        compiler_params=pltpu.CompilerParams(use_tc_tiling_on_sc=False),
    )


def _zero_out(rows_v, out_sh, zeros128, nstart, s):
    pltpu.sync_copy(zeros128, rows_v)
    for z in range(9):
        pltpu.sync_copy(rows_v.at[pl.ds(0, 64)],
                        out_sh.at[pl.ds(nstart + z * 64, 64)])
    pltpu.sync_copy(rows_v.at[pl.ds(0, 48)],
                    out_sh.at[pl.ds(nstart + 576, 48)])

    @pl.when(s < 2)
    def _():
        pltpu.sync_copy(rows_v.at[pl.ds(0, 8)],
                        out_sh.at[pl.ds(nstart + 624, 8)])


def _edge_chunk(ytab, src_v, dst_v, rows2, out_sh, sem_g, sem_s, nrows):
    gprev = pltpu.async_copy(ytab.at[src_v.at[0]], rows2[0], sem_g)
    sdesc = []
    for u in range(nrows):
        if u + 1 < nrows:
            if u >= 1:
                sdesc[u - 1].wait()
            gnext = pltpu.async_copy(ytab.at[src_v.at[u + 1]],
                                     rows2[(u + 1) % 2], sem_g)
        gprev.wait()
        sdesc.append(pltpu.async_copy(rows2[u % 2],
                                      out_sh.at[dst_v.at[u]],
                                      sem_s, add=True))
        if u + 1 < nrows:
            gprev = gnext
    if nrows >= 2:
        sdesc[nrows - 2].wait()
    sdesc[nrows - 1].wait()


def _writeout(out_sh, s_out, coff, nstart, s):
    pltpu.sync_copy(out_sh.at[pl.ds(nstart, 624)],
                    s_out.at[pl.ds(coff + nstart, 624)])

    @pl.when(s < 2)
    def _():
        pltpu.sync_copy(out_sh.at[pl.ds(nstart + 624, 8)],
                        s_out.at[pl.ds(coff + nstart + 624, 8)])


def _scatA_body(y1, srcE, dstE, zeros128, s_out,
                src_v, dst_v, rows_a, rows_b, out_sh, sem_g, sem_s):
    c = lax.axis_index("c")
    s = lax.axis_index("s")
    w = c * 16 + s
    start = w * 312 + 8 * jnp.minimum(w, 2)
    nstart, _ = _node_split(s)
    rows2 = (rows_a, rows_b)

    _zero_out(rows_a, out_sh, zeros128, nstart, s)
    plsc.subcore_barrier()

    def full_chunk(k, carry):
        base = start + k * 24
        pltpu.sync_copy(srcE.at[pl.ds(base, 24)], src_v)
        pltpu.sync_copy(dstE.at[pl.ds(base, 24)], dst_v)
        _edge_chunk(y1, src_v, dst_v, rows2, out_sh, sem_g, sem_s, 24)
        return carry

    lax.fori_loop(0, 13, full_chunk, 0)

    @pl.when(w < 2)
    def _():
        base = start + 312
        pltpu.sync_copy(srcE.at[pl.ds(base, 8)], src_v.at[pl.ds(0, 8)])
        pltpu.sync_copy(dstE.at[pl.ds(base, 8)], dst_v.at[pl.ds(0, 8)])
        _edge_chunk(y1, src_v, dst_v, rows2, out_sh, sem_g, sem_s, 8)

    plsc.subcore_barrier()
    _writeout(out_sh, s_out, c * _N, nstart, s)


@functools.cache
def _make_scatA():
    return pl.kernel(
        _scatA_body,
        out_type=jax.ShapeDtypeStruct((2 * _N, 128), jnp.float32),
        mesh=plsc.VectorSubcoreMesh(core_axis_name="c", subcore_axis_name="s"),
        scratch_types=[
            pltpu.VMEM((24, 64), jnp.int32),
            pltpu.VMEM((24, 64), jnp.int32),
            pltpu.VMEM((64, 128), jnp.float32),
            pltpu.VMEM((64, 128), jnp.float32),
            pltpu.VMEM_SHARED((_N, 128), jnp.float32),
            pltpu.SemaphoreType.DMA,
            pltpu.SemaphoreType.DMA,
        ],
        compiler_params=pltpu.CompilerParams(use_tc_tiling_on_sc=False),
    )


def _scatB_body(y2f, srcE2, dstE, zeros128, s_out,
                src_v, dst_v, rows_a, rows_b, out_sh, sem_g, sem_s):
    c = lax.axis_index("c")
    s = lax.axis_index("s")
    start, _ = _node_split(s)
    nstart = start
    rows2 = (rows_a, rows_b)

    _zero_out(rows_a, out_sh, zeros128, nstart, s)
    plsc.subcore_barrier()

    def full_chunk(k, carry):
        base = start + k * 24
        pltpu.sync_copy(srcE2.at[c, pl.ds(base, 24)], src_v)
        pltpu.sync_copy(dstE.at[pl.ds(base, 24)], dst_v)
        _edge_chunk(y2f, src_v, dst_v, rows2, out_sh, sem_g, sem_s, 24)
        return carry

    lax.fori_loop(0, 26, full_chunk, 0)

    @pl.when(s < 2)
    def _():
        base = start + 624
        pltpu.sync_copy(srcE2.at[c, pl.ds(base, 8)], src_v.at[pl.ds(0, 8)])
        pltpu.sync_copy(dstE.at[pl.ds(base, 8)], dst_v.at[pl.ds(0, 8)])
        _edge_chunk(y2f, src_v, dst_v, rows2, out_sh, sem_g, sem_s, 8)

    plsc.subcore_barrier()
    _writeout(out_sh, s_out, c * _N, nstart, s)


@functools.cache
def _make_scatB():
    return pl.kernel(
        _scatB_body,
        out_type=jax.ShapeDtypeStruct((2 * _N, 128), jnp.float32),
        mesh=plsc.VectorSubcoreMesh(core_axis_name="c", subcore_axis_name="s"),
        scratch_types=[
            pltpu.VMEM((24, 64), jnp.int32),
            pltpu.VMEM((24, 64), jnp.int32),
            pltpu.VMEM((64, 128), jnp.float32),
            pltpu.VMEM((64, 128), jnp.float32),
            pltpu.VMEM_SHARED((_N, 128), jnp.float32),
            pltpu.SemaphoreType.DMA,
            pltpu.SemaphoreType.DMA,
        ],
        compiler_params=pltpu.CompilerParams(use_tc_tiling_on_sc=False),
    )


_R = 1000


def _y1_body(x_ref, w1_ref, degp_ref, y1_ref, dis_ref):
    deg = degp_ref[0][:, 0:1] + degp_ref[1][:, 0:1] + 1.0
    dis = lax.rsqrt(deg)
    xw = jnp.dot(x_ref[...], w1_ref[...], preferred_element_type=jnp.float32)
    y1_ref[...] = xw * dis
    dis_ref[...] = dis


def _y1_call(x, W1, degp):
    return pl.pallas_call(
        _y1_body,
        grid=(_N // _R,),
        in_specs=[
            pl.BlockSpec((_R, 128), lambda i: (i, 0)),
            pl.BlockSpec((128, 128), lambda i: (0, 0)),
            pl.BlockSpec((2, _R, 16), lambda i: (0, i, 0)),
        ],
        out_specs=[
            pl.BlockSpec((_R, 128), lambda i: (i, 0)),
            pl.BlockSpec((_R, 1), lambda i: (i, 0)),
        ],
        out_shape=[
            jax.ShapeDtypeStruct((_N, 128), jnp.float32),
            jax.ShapeDtypeStruct((_N, 1), jnp.float32),
        ],
    )(x, W1, degp)


def _mid_body(s1_ref, y1_ref, dis_ref, b1_ref, w2_ref, y2_ref):
    dis = dis_ref[...]
    pre = s1_ref[0] + s1_ref[1] + y1_ref[...]
    h = jnp.maximum(pre * dis + b1_ref[...], 0.0)
    xw2 = jnp.dot(h, w2_ref[...], preferred_element_type=jnp.float32)
    y2 = xw2 * dis
    y2_ref[0] = y2[:, :128]
    y2_ref[1] = y2[:, 128:]


def _mid_call(s1p, y1, dis, b1r, W2):
    return pl.pallas_call(
        _mid_body,
        grid=(_N // _R,),
        in_specs=[
            pl.BlockSpec((2, _R, 128), lambda i: (0, i, 0)),
            pl.BlockSpec((_R, 128), lambda i: (i, 0)),
            pl.BlockSpec((_R, 1), lambda i: (i, 0)),
            pl.BlockSpec((1, 128), lambda i: (0, 0)),
            pl.BlockSpec((128, 256), lambda i: (0, 0)),
        ],
        out_specs=pl.BlockSpec((2, _R, 128), lambda i: (0, i, 0)),
        out_shape=jax.ShapeDtypeStruct((2, _N, 128), jnp.float32),
    )(s1p, y1, dis, b1r, W2)


def _fin_body(s2_ref, y2_ref, dis_ref, b2_ref, wl_ref, bl_ref, out_ref):
    dis = dis_ref[...]
    pre = jnp.concatenate(
        [s2_ref[0] + y2_ref[0], s2_ref[1] + y2_ref[1]], axis=1)
    h2 = pre * dis + b2_ref[...]
    out_ref[...] = (jnp.dot(h2, wl_ref[...],
                            preferred_element_type=jnp.float32)
                    + bl_ref[...])


def _fin_call(s2f, y2, dis, b2r, WL, bLr):
    return pl.pallas_call(
        _fin_body,
        grid=(_N // _R,),
        in_specs=[
            pl.BlockSpec((2, _R, 128), lambda i: (0, i, 0)),
            pl.BlockSpec((2, _R, 128), lambda i: (0, i, 0)),
            pl.BlockSpec((_R, 1), lambda i: (i, 0)),
            pl.BlockSpec((1, 256), lambda i: (0, 0)),
            pl.BlockSpec((256, 16), lambda i: (0, 0)),
            pl.BlockSpec((1, 16), lambda i: (0, 0)),
        ],
        out_specs=pl.BlockSpec((_R, 16), lambda i: (i, 0)),
        out_shape=jax.ShapeDtypeStruct((_N, 16), jnp.float32),
    )(s2f, y2, dis, b2r, WL, bLr)


def kernel(x, edge_index, W1, b1, W2, b2, WL, bL):
    srcE = edge_index[0].reshape(_ER64, 64)
    dstE = edge_index[1].reshape(_ER64, 64)
    srcE2 = jnp.stack([srcE, srcE + _N])
    dstR = edge_index[1].reshape(_ER, 128)
    ones128 = jnp.ones((128, 16), jnp.float32)
    zeros16 = jnp.zeros((156, 16), jnp.float32)
    zeros128 = jnp.zeros((64, 128), jnp.float32)

    degp = _deg_call()(dstR, ones128, zeros16)
    y1, dis = _y1_call(x, W1, degp)
    s1p = _make_scatA()(y1, srcE, dstE, zeros128).reshape(2, _N, 128)
    y2 = _mid_call(s1p, y1, dis, b1.reshape(1, 128), W2)
    s2f = _make_scatB()(y2.reshape(2 * _N, 128), srcE2, dstE,
                        zeros128).reshape(2, _N, 128)
    return _fin_call(s2f, y2, dis, b2.reshape(1, 256), WL, bL.reshape(1, 16))
